# baseline (speedup 1.0000x reference)
import functools

import jax
import jax.numpy as jnp
from jax import lax
from jax.experimental import pallas as pl
from jax.experimental.pallas import tpu as pltpu
from jax.experimental.pallas import tpu_sc as plsc

H = 768
NTOK = 8192
NLBL = 64

NC = 2
NS = 16
NW = NC * NS
PER_W = NTOK // NW
CH = 128

BT = 256
NB = NTOK // BT

def _I0():
    return jnp.int32(0)


def _table_body(lf_ref, w2_ref, b_ref, t_ref):
    t_ref[...] = (
        jnp.dot(lf_ref[...], w2_ref[...], preferred_element_type=jnp.float32)
        + b_ref[...]
    )


def _project_table(label_feature, w2, b2d):
    return pl.pallas_call(
        _table_body,
        out_shape=jax.ShapeDtypeStruct((NLBL, H), jnp.float32),
    )(label_feature, w2, b2d)


@functools.cache
def _make_sc_gather():
    @functools.partial(
        pl.kernel,
        out_type=jax.ShapeDtypeStruct((NTOK, H), jnp.float32),
        mesh=plsc.VectorSubcoreMesh(
            core_axis_name="c",
            subcore_axis_name="s",
            num_cores=NC,
            num_subcores=NS,
        ),
        scratch_types=[
            pltpu.VMEM((CH,), jnp.int32),
            pltpu.VMEM((CH, H), jnp.float32),
            pltpu.SemaphoreType.DMA,
        ],
    )
    def _sc_gather(lbl_hbm, t_hbm, g_hbm, idx_v, rows_v, sem):
        wid = lax.axis_index("s") * NC + lax.axis_index("c")
        base = wid * PER_W
        for c in range(PER_W // CH):
            off = base + c * CH
            pltpu.sync_copy(lbl_hbm.at[pl.ds(off, CH)], idx_v)
            for j in range(CH // 16):
                v = idx_v[pl.ds(j * 16, 16)]
                idx_v[pl.ds(j * 16, 16)] = jnp.maximum(v - 1, 0)
            pltpu.async_copy(t_hbm.at[idx_v], rows_v, sem).wait()
            pltpu.sync_copy(rows_v, g_hbm.at[pl.ds(off, CH)])

    return _sc_gather


def _merge_body(x_ref, g_ref, m_ref, w1_ref, o_ref):
    x = x_ref[...]
    y = jnp.dot(x, w1_ref[...], preferred_element_type=jnp.float32)
    o_ref[...] = x + m_ref[...] * (y + g_ref[...] - x)


def _merge(x, g, m, w1):
    return pl.pallas_call(
        _merge_body,
        grid=(NB,),
        in_specs=[
            pl.BlockSpec((BT, H), lambda i: (i, _I0())),
            pl.BlockSpec((BT, H), lambda i: (i, _I0())),
            pl.BlockSpec((BT, 1), lambda i: (i, _I0())),
            pl.BlockSpec((H, H), lambda i: (_I0(), _I0())),
        ],
        out_specs=pl.BlockSpec((BT, H), lambda i: (i, _I0())),
        out_shape=jax.ShapeDtypeStruct((NTOK, H), jnp.float32),
        compiler_params=pltpu.CompilerParams(
            dimension_semantics=("arbitrary",)
        ),
    )(x, g, m, w1)


def kernel(com_features, labels, label_feature, W_merge, b_merge):
    x = com_features.reshape(NTOK, H)
    lbl = labels.reshape(NTOK).astype(jnp.int32)
    w1 = W_merge[:H]
    w2 = W_merge[H:]
    t = _project_table(label_feature, w2, b_merge.reshape(1, H))
    g = _make_sc_gather()(lbl, t)
    m = (lbl != 0).astype(jnp.float32).reshape(NTOK, 1)
    out = _merge(x, g, m, w1)
    return out.reshape(com_features.shape)
